# baseline (speedup 1.0000x reference)
import functools

import jax
import jax.numpy as jnp
from jax import lax
from jax.experimental import pallas as pl
from jax.experimental.pallas import tpu as pltpu
from jax.experimental.pallas import tpu_sc as plsc

LANES = 16
NCORES = 2
NSUB = 16
NWORKERS = NCORES * NSUB
NCLS = 1000
WROWS = 32
NSPLIT = 2


def _sc_gather_cols(ub_flat, lb_flat, labels_part):
    B = labels_part.shape[0]
    mesh = plsc.VectorSubcoreMesh(core_axis_name="c", subcore_axis_name="s")

    @functools.partial(
        pl.kernel,
        out_type=jax.ShapeDtypeStruct((NCLS, 2, B // 128, 128), jnp.float32),
        mesh=mesh,
        scratch_types=[
            pltpu.VMEM((B,), jnp.int32),
            pltpu.VMEM((WROWS * NCLS,), jnp.float32),
            pltpu.VMEM((WROWS * NCLS,), jnp.float32),
            pltpu.VMEM((B // 128, 128), jnp.float32),
            pltpu.VMEM((B // 128, 128), jnp.float32),
            pltpu.VMEM((B // 128, 128), jnp.float32),
            pltpu.VMEM((B // 128, 128), jnp.float32),
            pltpu.SemaphoreType.DMA,
            pltpu.SemaphoreType.DMA,
        ],
        compiler_params=pltpu.CompilerParams(needs_layout_passes=False),
    )
    def k(ub_hbm, lb_hbm, lbl_hbm, out_t,
          lbl_v, ubr, lbr, oub0, oub1, olb0, olb1, sem_in, sem_out):
        wid = lax.axis_index("s") * NCORES + lax.axis_index("c")
        start = lax.shift_right_logical(wid * 125, 2)
        flat_start = start * NCLS
        pltpu.sync_copy(lbl_hbm, lbl_v)
        cin_u = pltpu.async_copy(
            ub_hbm.at[pl.ds(flat_start, WROWS * NCLS)], ubr, sem_in)
        cin_l = pltpu.async_copy(
            lb_hbm.at[pl.ds(flat_start, WROWS * NCLS)], lbr, sem_in)
        cin_u.wait()
        cin_l.wait()

        obufs = ((oub0, olb0), (oub1, olb1))
        pending = []
        nsub = B // 128
        for r in range(WROWS):
            ob_u, ob_l = obufs[r % 2]
            if r >= 2:
                pending[2 * (r - 2)].wait()
                pending[2 * (r - 2) + 1].wait()
            row_base = jnp.full((LANES,), r * NCLS, jnp.int32)

            @plsc.parallel_loop(0, nsub)
            def sbody(s, ob_u=ob_u, ob_l=ob_l, row_base=row_base):
                soff = pl.multiple_of(s * 128, 128)
                for kk in range(8):
                    idx = lbl_v[pl.ds(soff + kk * LANES, LANES)] + row_base
                    ob_u[s, pl.ds(kk * LANES, LANES)] = plsc.load_gather(
                        ubr, [idx])
                    ob_l[s, pl.ds(kk * LANES, LANES)] = plsc.load_gather(
                        lbr, [idx])

            pending.append(
                pltpu.async_copy(ob_u, out_t.at[start + r, 0], sem_out))
            pending.append(
                pltpu.async_copy(ob_l, out_t.at[start + r, 1], sem_out))
        for cp in pending[2 * (WROWS - 2):]:
            cp.wait()

    return k(ub_flat, lb_flat, labels_part)


def _tc_loss(labels, f3, t3, denom):
    n_rows = f3.shape[0]
    nsub = f3.shape[1]
    block_rows = 512
    grid = (n_rows // block_rows,)

    def body(lbl_ref, f_ref, t_hbm, out_ref, t_v, acc_ref, sem):
        i = pl.program_id(0)

        @pl.when(i == 0)
        def _():
            ct = pltpu.make_async_copy(t_hbm, t_v, sem)
            ct.start()
            ct.wait()
            acc_ref[...] = jnp.zeros_like(acc_ref)

        base = i * block_rows

        def grp(g, acc):
            r0 = g * 8
            terms = []
            for k in range(8):
                l = lbl_ref[base + r0 + k]
                f = f_ref[r0 + k]
                terms.append(jnp.maximum(t_v[l, 1] - f, 0.0)
                             + jnp.maximum(f - t_v[l, 0], 0.0))
            t01 = terms[0] + terms[1]
            t23 = terms[2] + terms[3]
            t45 = terms[4] + terms[5]
            t67 = terms[6] + terms[7]
            return acc + ((t01 + t23) + (t45 + t67))

        acc_ref[...] = lax.fori_loop(0, block_rows // 8, grp, acc_ref[...])

        @pl.when(i == grid[0] - 1)
        def _():
            out_ref[0] = jnp.sum(acc_ref[...]) / denom

    grid_spec = pltpu.PrefetchScalarGridSpec(
        num_scalar_prefetch=1,
        grid=grid,
        in_specs=[
            pl.BlockSpec((block_rows, nsub, 128), lambda i, lbl: (i, 0, 0)),
            pl.BlockSpec(memory_space=pltpu.MemorySpace.HBM),
        ],
        out_specs=pl.BlockSpec(memory_space=pltpu.MemorySpace.SMEM),
        scratch_shapes=[
            pltpu.VMEM((NCLS, 2, nsub, 128), jnp.float32),
            pltpu.VMEM((nsub, 128), jnp.float32),
            pltpu.SemaphoreType.DMA,
        ],
    )
    return pl.pallas_call(
        body,
        grid_spec=grid_spec,
        out_shape=jax.ShapeDtypeStruct((1,), jnp.float32),
    )(labels, f3, t3)


def kernel(FSR_Mat, labels, class_smi_UB, class_smi_LB):
    b = FSR_Mat.shape[0]
    rp = b // NSPLIT
    t3 = _sc_gather_cols(class_smi_UB.reshape(-1), class_smi_LB.reshape(-1),
                         labels)
    denom = float(b) * float(b)
    losses = []
    for h in range(NSPLIT):
        f_h = FSR_Mat[h * rp:(h + 1) * rp, :].reshape(rp, b // 128, 128)
        lbl_h = labels[h * rp:(h + 1) * rp]
        losses.append(_tc_loss(lbl_h, f_h, t3, denom)[0])
    out = losses[0]
    for x in losses[1:]:
        out = out + x
    return out


# speedup vs baseline: 1.2246x; 1.2246x over previous
import functools

import jax
import jax.numpy as jnp
from jax import lax
from jax.experimental import pallas as pl
from jax.experimental.pallas import tpu as pltpu
from jax.experimental.pallas import tpu_sc as plsc

LANES = 16
NCORES = 2
NSUB = 16
NWORKERS = NCORES * NSUB
NCLS = 1000
WROWS = 32
NSPLIT = 1


def _sc_gather_cols(ub_flat, lb_flat, labels_part):
    B = labels_part.shape[0]
    mesh = plsc.VectorSubcoreMesh(core_axis_name="c", subcore_axis_name="s")

    @functools.partial(
        pl.kernel,
        out_type=jax.ShapeDtypeStruct((NCLS, 2, B // 128, 128), jnp.float32),
        mesh=mesh,
        scratch_types=[
            pltpu.VMEM((B,), jnp.int32),
            pltpu.VMEM((WROWS * NCLS,), jnp.float32),
            pltpu.VMEM((WROWS * NCLS,), jnp.float32),
            pltpu.VMEM((B // 128, 128), jnp.float32),
            pltpu.VMEM((B // 128, 128), jnp.float32),
            pltpu.VMEM((B // 128, 128), jnp.float32),
            pltpu.VMEM((B // 128, 128), jnp.float32),
            pltpu.SemaphoreType.DMA,
            pltpu.SemaphoreType.DMA,
        ],
        compiler_params=pltpu.CompilerParams(needs_layout_passes=False),
    )
    def k(ub_hbm, lb_hbm, lbl_hbm, out_t,
          lbl_v, ubr, lbr, oub0, oub1, olb0, olb1, sem_in, sem_out):
        wid = lax.axis_index("s") * NCORES + lax.axis_index("c")
        start = lax.shift_right_logical(wid * 125, 2)
        flat_start = start * NCLS
        pltpu.sync_copy(lbl_hbm, lbl_v)
        cin_u = pltpu.async_copy(
            ub_hbm.at[pl.ds(flat_start, WROWS * NCLS)], ubr, sem_in)
        cin_l = pltpu.async_copy(
            lb_hbm.at[pl.ds(flat_start, WROWS * NCLS)], lbr, sem_in)
        cin_u.wait()
        cin_l.wait()

        obufs = ((oub0, olb0), (oub1, olb1))
        pending = []
        nsub = B // 128
        for r in range(WROWS):
            ob_u, ob_l = obufs[r % 2]
            if r >= 2:
                pending[2 * (r - 2)].wait()
                pending[2 * (r - 2) + 1].wait()
            row_base = jnp.full((LANES,), r * NCLS, jnp.int32)

            @plsc.parallel_loop(0, nsub)
            def sbody(s, ob_u=ob_u, ob_l=ob_l, row_base=row_base):
                soff = pl.multiple_of(s * 128, 128)
                for kk in range(8):
                    idx = lbl_v[pl.ds(soff + kk * LANES, LANES)] + row_base
                    ob_u[s, pl.ds(kk * LANES, LANES)] = plsc.load_gather(
                        ubr, [idx])
                    ob_l[s, pl.ds(kk * LANES, LANES)] = plsc.load_gather(
                        lbr, [idx])

            pending.append(
                pltpu.async_copy(ob_u, out_t.at[start + r, 0], sem_out))
            pending.append(
                pltpu.async_copy(ob_l, out_t.at[start + r, 1], sem_out))
        for cp in pending[2 * (WROWS - 2):]:
            cp.wait()

    return k(ub_flat, lb_flat, labels_part)


def _tc_loss(labels, f3, t3, denom):
    n_rows = f3.shape[0]
    nsub = f3.shape[1]
    block_rows = 512
    grid = (n_rows // block_rows,)

    def body(lbl_ref, f_ref, t_hbm, out_ref, t_v, acc_ref, sem):
        i = pl.program_id(0)

        @pl.when(i == 0)
        def _():
            ct = pltpu.make_async_copy(t_hbm, t_v, sem)
            ct.start()
            ct.wait()
            acc_ref[...] = jnp.zeros_like(acc_ref)

        base = i * block_rows

        def grp(g, acc):
            r0 = g * 8
            terms = []
            for k in range(8):
                l = lbl_ref[base + r0 + k]
                f = f_ref[r0 + k]
                terms.append(jnp.maximum(t_v[l, 1] - f, 0.0)
                             + jnp.maximum(f - t_v[l, 0], 0.0))
            t01 = terms[0] + terms[1]
            t23 = terms[2] + terms[3]
            t45 = terms[4] + terms[5]
            t67 = terms[6] + terms[7]
            return acc + ((t01 + t23) + (t45 + t67))

        acc_ref[...] = lax.fori_loop(0, block_rows // 8, grp, acc_ref[...])

        @pl.when(i == grid[0] - 1)
        def _():
            out_ref[0] = jnp.sum(acc_ref[...]) / denom

    grid_spec = pltpu.PrefetchScalarGridSpec(
        num_scalar_prefetch=1,
        grid=grid,
        in_specs=[
            pl.BlockSpec((block_rows, nsub, 128), lambda i, lbl: (i, 0, 0)),
            pl.BlockSpec(memory_space=pltpu.MemorySpace.HBM),
        ],
        out_specs=pl.BlockSpec(memory_space=pltpu.MemorySpace.SMEM),
        scratch_shapes=[
            pltpu.VMEM((NCLS, 2, nsub, 128), jnp.float32),
            pltpu.VMEM((nsub, 128), jnp.float32),
            pltpu.SemaphoreType.DMA,
        ],
    )
    return pl.pallas_call(
        body,
        grid_spec=grid_spec,
        out_shape=jax.ShapeDtypeStruct((1,), jnp.float32),
    )(labels, f3, t3)


def kernel(FSR_Mat, labels, class_smi_UB, class_smi_LB):
    b = FSR_Mat.shape[0]
    bp = b // NSPLIT
    ub_flat = class_smi_UB.reshape(-1)
    lb_flat = class_smi_LB.reshape(-1)
    t_parts = [
        _sc_gather_cols(ub_flat, lb_flat, labels[h * bp:(h + 1) * bp])
        for h in range(NSPLIT)
    ]
    f_parts = [
        FSR_Mat[:, h * bp:(h + 1) * bp].reshape(b, bp // 128, 128)
        for h in range(NSPLIT)
    ]
    denom = float(b) * float(b)
    losses = [
        _tc_loss(labels, f_parts[h], t_parts[h], denom)[0]
        for h in range(NSPLIT)
    ]
    out = losses[0]
    for x in losses[1:]:
        out = out + x
    return out
